# baseline (speedup 1.0000x reference)
import functools
import jax
import jax.numpy as jnp
from jax import lax
from jax.experimental import pallas as pl
from jax.experimental.pallas import tpu as pltpu
from jax.experimental.pallas import tpu_sc as plsc

D = 32768
M = 8
H = 64
K = 327
GAMMA = 0.105
BD = 2048
NB = D // BD

_info = plsc.get_sparse_core_info()
NC, NS, L = _info.num_cores, _info.num_subcores, _info.num_lanes
NW = NC * NS
COLS = D // NW


def _fused_kernel(
    ct_ref, W1_ref, b1_ref, W2_ref, b2_ref, W3_ref, b3_ref, Q_ref,
    wr_out_ref, thr_ref, acc_ref, h2_ref, ax_ref,
):
    p = pl.program_id(0)
    i = pl.program_id(1)
    ct_blk = ct_ref[0, pl.ds(i * BD, BD)].reshape(1, BD)

    @pl.when(p == 0)
    def _phase0():
        @pl.when(i == 0)
        def _init():
            acc_ref[...] = jnp.zeros_like(acc_ref)

        parts = [
            jnp.dot(ct_blk, W1_ref[m], preferred_element_type=jnp.float32)
            for m in range(M)
        ]
        acc_ref[...] += jnp.concatenate(parts, axis=0)

        @pl.when(i == NB - 1)
        def _h2():
            h1 = jnp.maximum(acc_ref[...] + b1_ref[...], 0.0)
            h2s = [
                jnp.dot(h1[m : m + 1], W2_ref[m], preferred_element_type=jnp.float32)
                for m in range(M)
            ]
            h2_ref[...] = jnp.maximum(jnp.concatenate(h2s, axis=0) + b2_ref[...], 0.0)

    @pl.when(p == 1)
    def _phase1():
        h2 = h2_ref[...]
        vms = [
            jnp.dot(h2[m : m + 1], W3_ref[m], preferred_element_type=jnp.float32)
            for m in range(M)
        ]
        vm = jnp.concatenate(vms, axis=0) + b3_ref[...]
        gate = jax.nn.sigmoid(Q_ref[...] * ct_blk)
        w = gate * vm
        wr_out_ref[...] = w
        ax_ref[:, pl.ds(i * BD, BD)] = jax.lax.bitcast_convert_type(
            w, jnp.int32
        ) & jnp.int32(0x7FFFFFFF)

        @pl.when(i == NB - 1)
        def _finish():
            ax = ax_ref[...]

            def body(_, lohi):
                lo, hi = lohi
                mid = lo + ((hi - lo + 1) >> 1)
                cnt = jnp.sum((ax >= mid).astype(jnp.int32), axis=1, keepdims=True)
                ge = cnt >= K
                return jnp.where(ge, mid, lo), jnp.where(ge, hi, mid - 1)

            lo0 = jnp.zeros((M, 1), jnp.int32)
            hi0 = jnp.full((M, 1), 0x7F800000, jnp.int32)
            thr, _ = jax.lax.fori_loop(0, 31, body, (lo0, hi0))
            thr_ref[...] = jnp.broadcast_to(
                jax.lax.bitcast_convert_type(thr, jnp.float32), (M, 128)
            )


def _make_sc_scatter():
    mesh = plsc.VectorSubcoreMesh(core_axis_name="c", subcore_axis_name="s")

    @functools.partial(
        pl.kernel,
        mesh=mesh,
        out_type=jax.ShapeDtypeStruct((D,), jnp.float32),
        scratch_types=[
            pltpu.VMEM((M, COLS), jnp.float32),
            pltpu.VMEM((COLS,), jnp.float32),
            pltpu.VMEM((M * L,), jnp.float32),
            pltpu.VMEM((COLS,), jnp.float32),
        ],
    )
    def sc_scatter(wr_hbm, thr_hbm, ct_hbm, out_hbm, buf_w, buf_ct, buf_t, buf_o):
        wid = lax.axis_index("s") * NC + lax.axis_index("c")
        base = wid * COLS
        for m in range(M):
            pltpu.sync_copy(wr_hbm.at[pl.ds(m * D + base, COLS)], buf_w.at[m])
        pltpu.sync_copy(ct_hbm.at[pl.ds(base, COLS)], buf_ct)
        pltpu.sync_copy(thr_hbm, buf_t)
        tvs = [buf_t[pl.ds(m * L, L)] for m in range(M)]
        for c in range(COLS // L):
            j = c * L
            av = (1.0 - GAMMA) * buf_ct[pl.ds(j, L)]
            for m in range(M):
                wv = buf_w[m, pl.ds(j, L)]
                keep = jnp.abs(wv) >= tvs[m]
                av = av + jnp.where(keep, wv, 0.0)
            buf_o[pl.ds(j, L)] = av
        pltpu.sync_copy(buf_o, out_hbm.at[pl.ds(base, COLS)])

    return sc_scatter


def kernel(ct, W1, b1, W2, b2, W3, b3, Q):
    ct2 = ct.reshape(1, D)
    wr, thr = pl.pallas_call(
        _fused_kernel,
        grid=(2, NB),
        in_specs=[
            pl.BlockSpec((1, D), lambda p, i: (0, 0)),
            pl.BlockSpec((M, BD, H), lambda p, i: (0, jnp.where(p == 0, i, NB - 1), 0)),
            pl.BlockSpec((M, H), lambda p, i: (0, 0)),
            pl.BlockSpec((M, H, H), lambda p, i: (0, 0, 0)),
            pl.BlockSpec((M, H), lambda p, i: (0, 0)),
            pl.BlockSpec((M, H, BD), lambda p, i: (0, 0, jnp.where(p == 0, 0, i))),
            pl.BlockSpec((M, BD), lambda p, i: (0, jnp.where(p == 0, 0, i))),
            pl.BlockSpec((M, BD), lambda p, i: (0, jnp.where(p == 0, 0, i))),
        ],
        out_specs=[
            pl.BlockSpec((M, BD), lambda p, i: (0, jnp.where(p == 0, 0, i))),
            pl.BlockSpec((M, 128), lambda p, i: (0, 0)),
        ],
        out_shape=[
            jax.ShapeDtypeStruct((M, D), jnp.float32),
            jax.ShapeDtypeStruct((M, 128), jnp.float32),
        ],
        scratch_shapes=[
            pltpu.VMEM((M, H), jnp.float32),
            pltpu.VMEM((M, H), jnp.float32),
            pltpu.VMEM((M, D), jnp.int32),
        ],
        compiler_params=pltpu.CompilerParams(
            dimension_semantics=("arbitrary", "arbitrary"),
        ),
    )(ct2, W1, b1, W2, b2, W3, b3, Q)

    wr_flat = wr.reshape(M * D)
    thr16 = thr[:, :L].reshape(M * L)
    return _make_sc_scatter()(wr_flat, thr16, ct)


# speedup vs baseline: 1.0214x; 1.0214x over previous
import functools
import jax
import jax.numpy as jnp
from jax import lax
from jax.experimental import pallas as pl
from jax.experimental.pallas import tpu as pltpu
from jax.experimental.pallas import tpu_sc as plsc

D = 32768
M = 8
H = 64
K = 327
GAMMA = 0.105
BD = 2048
NB = D // BD

_info = plsc.get_sparse_core_info()
NC, NS, L = _info.num_cores, _info.num_subcores, _info.num_lanes
NW = NC * NS
COLS = D // NW


def _fused_kernel(
    ct_ref, W1_ref, b1_ref, W2_ref, b2_ref, W3_ref, b3_ref, Q_ref,
    wr_out_ref, thr_ref, acc_ref, h2_ref, ax_ref,
):
    p = pl.program_id(0)
    i = pl.program_id(1)
    ct_blk = ct_ref[0, pl.ds(i * BD, BD)].reshape(1, BD)

    @pl.when(p == 0)
    def _phase0():
        @pl.when(i == 0)
        def _init():
            acc_ref[...] = jnp.zeros_like(acc_ref)

        parts = [
            jnp.dot(ct_blk, W1_ref[m], preferred_element_type=jnp.float32)
            for m in range(M)
        ]
        acc_ref[...] += jnp.concatenate(parts, axis=0)

        @pl.when(i == NB - 1)
        def _h2():
            h1 = jnp.maximum(acc_ref[...] + b1_ref[...], 0.0)
            h2s = [
                jnp.dot(h1[m : m + 1], W2_ref[m], preferred_element_type=jnp.float32)
                for m in range(M)
            ]
            h2_ref[...] = jnp.maximum(jnp.concatenate(h2s, axis=0) + b2_ref[...], 0.0)

    @pl.when(p == 1)
    def _phase1():
        h2 = h2_ref[...]
        vms = [
            jnp.dot(h2[m : m + 1], W3_ref[m], preferred_element_type=jnp.float32)
            for m in range(M)
        ]
        vm = jnp.concatenate(vms, axis=0) + b3_ref[...]
        gate = jax.nn.sigmoid(Q_ref[...] * ct_blk)
        w = gate * vm
        wr_out_ref[...] = w
        ax_ref[:, pl.ds(i * BD, BD)] = jax.lax.bitcast_convert_type(
            w, jnp.int32
        ) & jnp.int32(0x7FFFFFFF)

        @pl.when(i == NB - 1)
        def _finish():
            ax = ax_ref[...]

            def body(_, lohi):
                lo, hi = lohi
                mid = lo + ((hi - lo + 1) >> 1)
                cnt = jnp.sum((ax >= mid).astype(jnp.int32), axis=1, keepdims=True)
                ge = cnt >= K
                return jnp.where(ge, mid, lo), jnp.where(ge, hi, mid - 1)

            lo0 = jnp.zeros((M, 1), jnp.int32)
            hi0 = jnp.full((M, 1), 0x7F800000, jnp.int32)
            thr, _ = jax.lax.fori_loop(0, 31, body, (lo0, hi0))
            thr_ref[...] = jnp.broadcast_to(
                jax.lax.bitcast_convert_type(thr, jnp.float32), (M, 128)
            )


def _make_sc_scatter():
    mesh = plsc.VectorSubcoreMesh(core_axis_name="c", subcore_axis_name="s")

    @functools.partial(
        pl.kernel,
        mesh=mesh,
        out_type=jax.ShapeDtypeStruct((D,), jnp.float32),
        scratch_types=[
            pltpu.VMEM((M, COLS), jnp.float32),
            pltpu.VMEM((COLS,), jnp.float32),
            pltpu.VMEM((M * L,), jnp.float32),
            pltpu.VMEM((COLS,), jnp.float32),
            pltpu.SemaphoreType.DMA,
        ],
    )
    def sc_scatter(wr_hbm, thr_hbm, ct_hbm, out_hbm, buf_w, buf_ct, buf_t, buf_o, sem):
        wid = lax.axis_index("s") * NC + lax.axis_index("c")
        base = wid * COLS
        handles = [
            pltpu.async_copy(wr_hbm.at[pl.ds(m * D + base, COLS)], buf_w.at[m], sem)
            for m in range(M)
        ]
        handles.append(pltpu.async_copy(ct_hbm.at[pl.ds(base, COLS)], buf_ct, sem))
        handles.append(pltpu.async_copy(thr_hbm, buf_t, sem))
        for hnd in handles:
            hnd.wait()
        tvs = [buf_t[pl.ds(m * L, L)] for m in range(M)]
        for c in range(COLS // L):
            j = c * L
            av = (1.0 - GAMMA) * buf_ct[pl.ds(j, L)]
            for m in range(M):
                wv = buf_w[m, pl.ds(j, L)]
                keep = jnp.abs(wv) >= tvs[m]
                av = av + jnp.where(keep, wv, 0.0)
            buf_o[pl.ds(j, L)] = av
        pltpu.sync_copy(buf_o, out_hbm.at[pl.ds(base, COLS)])

    return sc_scatter


def kernel(ct, W1, b1, W2, b2, W3, b3, Q):
    ct2 = ct.reshape(1, D)
    wr, thr = pl.pallas_call(
        _fused_kernel,
        grid=(2, NB),
        in_specs=[
            pl.BlockSpec((1, D), lambda p, i: (0, 0)),
            pl.BlockSpec((M, BD, H), lambda p, i: (0, jnp.where(p == 0, i, NB - 1), 0)),
            pl.BlockSpec((M, H), lambda p, i: (0, 0)),
            pl.BlockSpec((M, H, H), lambda p, i: (0, 0, 0)),
            pl.BlockSpec((M, H), lambda p, i: (0, 0)),
            pl.BlockSpec((M, H, BD), lambda p, i: (0, 0, jnp.where(p == 0, 0, i))),
            pl.BlockSpec((M, BD), lambda p, i: (0, jnp.where(p == 0, 0, i))),
            pl.BlockSpec((M, BD), lambda p, i: (0, jnp.where(p == 0, 0, i))),
        ],
        out_specs=[
            pl.BlockSpec((M, BD), lambda p, i: (0, jnp.where(p == 0, 0, i))),
            pl.BlockSpec((M, 128), lambda p, i: (0, 0)),
        ],
        out_shape=[
            jax.ShapeDtypeStruct((M, D), jnp.float32),
            jax.ShapeDtypeStruct((M, 128), jnp.float32),
        ],
        scratch_shapes=[
            pltpu.VMEM((M, H), jnp.float32),
            pltpu.VMEM((M, H), jnp.float32),
            pltpu.VMEM((M, D), jnp.int32),
        ],
        compiler_params=pltpu.CompilerParams(
            dimension_semantics=("arbitrary", "arbitrary"),
        ),
    )(ct2, W1, b1, W2, b2, W3, b3, Q)

    wr_flat = wr.reshape(M * D)
    thr16 = thr[:, :L].reshape(M * L)
    return _make_sc_scatter()(wr_flat, thr16, ct)


# speedup vs baseline: 1.0223x; 1.0009x over previous
import functools
import jax
import jax.numpy as jnp
from jax import lax
from jax.experimental import pallas as pl
from jax.experimental.pallas import tpu as pltpu
from jax.experimental.pallas import tpu_sc as plsc

D = 32768
M = 8
H = 64
K = 327
GAMMA = 0.105
BD = 2048
NB = D // BD

NC, NS, L = 2, 16, 16
NW = NC * NS
COLS = D // NW


def _fused_kernel(
    ct_ref, W1_ref, b1_ref, W2_ref, b2_ref, W3_ref, b3_ref, Q_ref,
    wr_out_ref, thr_ref, acc_ref, h2_ref, ax_ref,
):
    p = pl.program_id(0)
    i = pl.program_id(1)
    ct_blk = ct_ref[0, pl.ds(i * BD, BD)].reshape(1, BD)

    @pl.when(p == 0)
    def _phase0():
        @pl.when(i == 0)
        def _init():
            acc_ref[...] = jnp.zeros_like(acc_ref)

        parts = [
            jnp.dot(ct_blk, W1_ref[m], preferred_element_type=jnp.float32)
            for m in range(M)
        ]
        acc_ref[...] += jnp.concatenate(parts, axis=0)

        @pl.when(i == NB - 1)
        def _h2():
            h1 = jnp.maximum(acc_ref[...] + b1_ref[...], 0.0)
            h2s = [
                jnp.dot(h1[m : m + 1], W2_ref[m], preferred_element_type=jnp.float32)
                for m in range(M)
            ]
            h2_ref[...] = jnp.maximum(jnp.concatenate(h2s, axis=0) + b2_ref[...], 0.0)

    @pl.when(p == 1)
    def _phase1():
        h2 = h2_ref[...]
        vms = [
            jnp.dot(h2[m : m + 1], W3_ref[m], preferred_element_type=jnp.float32)
            for m in range(M)
        ]
        vm = jnp.concatenate(vms, axis=0) + b3_ref[...]
        gate = jax.nn.sigmoid(Q_ref[...] * ct_blk)
        w = gate * vm
        wr_out_ref[...] = w
        ax_ref[:, pl.ds(i * BD, BD)] = jax.lax.bitcast_convert_type(
            w, jnp.int32
        ) & jnp.int32(0x7FFFFFFF)

        @pl.when(i == NB - 1)
        def _finish():
            ax = ax_ref[...]

            def body(_, lohi):
                lo, hi = lohi
                mid = lo + ((hi - lo + 1) >> 1)
                cnt = jnp.sum((ax >= mid).astype(jnp.int32), axis=1, keepdims=True)
                ge = cnt >= K
                return jnp.where(ge, mid, lo), jnp.where(ge, hi, mid - 1)

            lo0 = jnp.zeros((M, 1), jnp.int32)
            hi0 = jnp.full((M, 1), 0x7F800000, jnp.int32)
            thr, _ = jax.lax.fori_loop(0, 31, body, (lo0, hi0))
            thr_ref[...] = jnp.broadcast_to(
                jax.lax.bitcast_convert_type(thr, jnp.float32), (M, 128)
            )


def _make_sc_scatter():
    mesh = plsc.VectorSubcoreMesh(core_axis_name="c", subcore_axis_name="s")

    @functools.partial(
        pl.kernel,
        mesh=mesh,
        out_type=jax.ShapeDtypeStruct((D,), jnp.float32),
        scratch_types=[
            pltpu.VMEM((M, COLS), jnp.float32),
            pltpu.VMEM((COLS,), jnp.float32),
            pltpu.VMEM((M * L,), jnp.float32),
            pltpu.VMEM((COLS,), jnp.float32),
            pltpu.SemaphoreType.DMA,
        ],
    )
    def sc_scatter(wr_hbm, thr_hbm, ct_hbm, out_hbm, buf_w, buf_ct, buf_t, buf_o, sem):
        wid = lax.axis_index("s") * NC + lax.axis_index("c")
        base = wid * COLS
        handles = [
            pltpu.async_copy(wr_hbm.at[pl.ds(m * D + base, COLS)], buf_w.at[m], sem)
            for m in range(M)
        ]
        handles.append(pltpu.async_copy(ct_hbm.at[pl.ds(base, COLS)], buf_ct, sem))
        handles.append(pltpu.async_copy(thr_hbm, buf_t, sem))
        for hnd in handles:
            hnd.wait()
        tvs = [buf_t[pl.ds(m * L, L)] for m in range(M)]
        for c in range(COLS // L):
            j = c * L
            av = (1.0 - GAMMA) * buf_ct[pl.ds(j, L)]
            for m in range(M):
                wv = buf_w[m, pl.ds(j, L)]
                keep = jnp.abs(wv) >= tvs[m]
                av = av + jnp.where(keep, wv, 0.0)
            buf_o[pl.ds(j, L)] = av
        pltpu.sync_copy(buf_o, out_hbm.at[pl.ds(base, COLS)])

    return sc_scatter


def kernel(ct, W1, b1, W2, b2, W3, b3, Q):
    ct2 = ct.reshape(1, D)
    wr, thr = pl.pallas_call(
        _fused_kernel,
        grid=(2, NB),
        in_specs=[
            pl.BlockSpec((1, D), lambda p, i: (0, 0)),
            pl.BlockSpec((M, BD, H), lambda p, i: (0, jnp.where(p == 0, i, NB - 1), 0)),
            pl.BlockSpec((M, H), lambda p, i: (0, 0)),
            pl.BlockSpec((M, H, H), lambda p, i: (0, 0, 0)),
            pl.BlockSpec((M, H), lambda p, i: (0, 0)),
            pl.BlockSpec((M, H, BD), lambda p, i: (0, 0, jnp.where(p == 0, 0, i))),
            pl.BlockSpec((M, BD), lambda p, i: (0, jnp.where(p == 0, 0, i))),
            pl.BlockSpec((M, BD), lambda p, i: (0, jnp.where(p == 0, 0, i))),
        ],
        out_specs=[
            pl.BlockSpec((M, BD), lambda p, i: (0, jnp.where(p == 0, 0, i))),
            pl.BlockSpec((M, 128), lambda p, i: (0, 0)),
        ],
        out_shape=[
            jax.ShapeDtypeStruct((M, D), jnp.float32),
            jax.ShapeDtypeStruct((M, 128), jnp.float32),
        ],
        scratch_shapes=[
            pltpu.VMEM((M, H), jnp.float32),
            pltpu.VMEM((M, H), jnp.float32),
            pltpu.VMEM((M, D), jnp.int32),
        ],
        compiler_params=pltpu.CompilerParams(
            dimension_semantics=("arbitrary", "arbitrary"),
        ),
    )(ct2, W1, b1, W2, b2, W3, b3, Q)

    wr_flat = wr.reshape(M * D)
    thr16 = thr[:, :L].reshape(M * L)
    return _make_sc_scatter()(wr_flat, thr16, ct)
